# baseline (speedup 1.0000x reference)
import functools

import jax
import jax.numpy as jnp
from jax.experimental import pallas as pl
from jax.experimental.pallas import tpu as pltpu

_CAPACITY = 0.5
_BLK_T = 256


def _select_topk_mask(g, k):
    b, t = g.shape
    bits = jax.lax.bitcast_convert_type(g, jnp.int32)

    def vbody(_, carry):
        lo, hi = carry
        mid = lo + ((hi - lo) >> 1)
        cnt = jnp.sum((bits >= mid).astype(jnp.int32), axis=1, keepdims=True)
        feas = cnt >= k
        return jnp.where(feas, mid, lo), jnp.where(feas, hi, mid)

    lo0 = jnp.zeros((b, 1), jnp.int32)
    hi0 = jnp.full((b, 1), jnp.int32(0x40000001))
    tbits, _ = jax.lax.fori_loop(0, 31, vbody, (lo0, hi0))

    gt = bits > tbits
    eq = bits == tbits
    need = k - jnp.sum(gt.astype(jnp.int32), axis=1, keepdims=True)
    iota = jax.lax.broadcasted_iota(jnp.int32, (b, t), 1)
    eqi = eq.astype(jnp.int32)

    def ibody(_, carry):
        lo, hi = carry
        mid = lo + ((hi - lo) >> 1)
        cnt = jnp.sum(eqi * (iota < mid).astype(jnp.int32), axis=1,
                      keepdims=True)
        geq = cnt >= need
        return jnp.where(geq, lo, mid), jnp.where(geq, mid, hi)

    lo0 = jnp.zeros((b, 1), jnp.int32)
    hi0 = jnp.full((b, 1), jnp.int32(t))
    _, jstar = jax.lax.fori_loop(0, 14, ibody, (lo0, hi0))

    return (gt | (eq & (iota < jstar))).astype(jnp.float32)


def _fused_kernel(scal_ref, a_ref, p_ref, g_ref, bin_ref, dst_s, dch_s,
                  *, inv_d, k, n_steps):
    i = pl.program_id(0)
    a = a_ref[...]
    p = p_ref[...]
    d = a - p
    blk = a.shape[1]
    dst_s[:, pl.ds(i * blk, blk)] = jnp.sum(a * a, axis=-1) * inv_d
    dch_s[:, pl.ds(i * blk, blk)] = jnp.sum(d * d, axis=-1) * inv_d

    @pl.when(i == n_steps - 1)
    def _():
        dst = dst_s[...]
        dch = dch_s[...]
        log_oce = scal_ref[0]
        m_cu = scal_ref[1]
        bce_pos = scal_ref[2]
        bcu_pos = scal_ref[3]

        ce = dst - (dch - log_oce)
        ma = jnp.mean(dst)
        cu = dst - m_cu * ma
        s_ce = jax.nn.sigmoid(bce_pos * ce)
        s_cu = jax.nn.sigmoid(bcu_pos * cu)
        g = s_ce + s_cu - s_ce * s_cu
        g_ref[...] = g
        bin_ref[...] = _select_topk_mask(g, k)


def kernel(actual_residual, predicted_residual, o_ce, m_cu, beta_ce, beta_cu):
    bv, tv, dv = actual_residual.shape
    k = max(1, int(tv * _CAPACITY))
    n_steps = tv // _BLK_T

    scal = jnp.stack([
        jnp.log(o_ce + 1e-10),
        m_cu,
        jax.nn.softplus(beta_ce),
        jax.nn.softplus(beta_cu),
    ]).astype(jnp.float32)

    g, binary = pl.pallas_call(
        functools.partial(_fused_kernel, inv_d=1.0 / dv, k=k,
                          n_steps=n_steps),
        grid=(n_steps,),
        in_specs=[
            pl.BlockSpec(memory_space=pltpu.SMEM),
            pl.BlockSpec((bv, _BLK_T, dv), lambda i: (0, i, 0)),
            pl.BlockSpec((bv, _BLK_T, dv), lambda i: (0, i, 0)),
        ],
        out_specs=[
            pl.BlockSpec((bv, tv), lambda i: (0, 0)),
            pl.BlockSpec((bv, tv), lambda i: (0, 0)),
        ],
        out_shape=[
            jax.ShapeDtypeStruct((bv, tv), jnp.float32),
            jax.ShapeDtypeStruct((bv, tv), jnp.float32),
        ],
        scratch_shapes=[
            pltpu.VMEM((bv, tv), jnp.float32),
            pltpu.VMEM((bv, tv), jnp.float32),
        ],
    )(scal, actual_residual, predicted_residual)

    return (g, binary)
